# baseline (speedup 1.0000x reference)
import functools

import jax
import jax.numpy as jnp
from jax.experimental import pallas as pl
from jax.experimental.pallas import tpu as pltpu


_EPS = 1e-12


def _rpq_body(x_ref, rp_ref, cbt_ref, iota_ref, out_ref, cbn_ref, cbsq_ref):
    i = pl.program_id(0)

    @pl.when(i == 0)
    def _init():
        cbt = cbt_ref[...]
        n = jnp.sqrt(jnp.sum(cbt * cbt, axis=0, keepdims=True))
        cbn = cbt / jnp.maximum(n, _EPS)
        cbn_ref[...] = cbn
        cbsq_ref[...] = jnp.sum(cbn * cbn, axis=0, keepdims=True)

    def matmul_stage(sl):
        proj = jnp.dot(x_ref[sl, :], rp_ref[...],
                       preferred_element_type=jnp.float32)
        norm = jnp.sqrt(jnp.sum(proj * proj, axis=1, keepdims=True))
        projn = proj / jnp.maximum(norm, _EPS)
        x_sq = jnp.sum(projn * projn, axis=1, keepdims=True)
        cm2 = jnp.dot(projn * (-2.0), cbn_ref[...],
                      preferred_element_type=jnp.float32)
        return x_sq, cm2

    def epilogue(x_sq, cm2, sl):
        d2 = (cbsq_ref[...] + x_sq) + cm2
        m2 = jnp.min(d2, axis=1, keepdims=True)
        s = jnp.sqrt(jnp.maximum(m2, 0.0))
        base = s * s
        bi = jax.lax.bitcast_convert_type(base, jnp.int32)
        x_hi = base
        for j in range(1, 7):
            xj = jax.lax.bitcast_convert_type(bi + j, jnp.float32)
            x_hi = jnp.where(jnp.sqrt(xj) == s, xj, x_hi)
        idx = jnp.min(jnp.where(d2 <= x_hi, iota_ref[...], jnp.inf), axis=1)
        out_ref[0, 0, sl] = idx.astype(jnp.int32)

    q = x_ref.shape[0] // 4
    sls = [slice(j * q, (j + 1) * q) for j in range(4)]
    r1 = matmul_stage(sls[0])
    r2 = matmul_stage(sls[1])
    epilogue(*r1, sls[0])
    r3 = matmul_stage(sls[2])
    epilogue(*r2, sls[1])
    r4 = matmul_stage(sls[3])
    epilogue(*r3, sls[2])
    epilogue(*r4, sls[3])


@functools.partial(jax.jit, static_argnames=())
def _rpq(x2, rp, cbt):
    bn, d = x2.shape
    k = cbt.shape[1]
    block_rows = 512
    nb = bn // block_rows
    iota = jnp.arange(k, dtype=jnp.float32).reshape(1, k)
    out = pl.pallas_call(
        _rpq_body,
        grid=(nb,),
        in_specs=[
            pl.BlockSpec((block_rows, d), lambda i: (i, 0)),
            pl.BlockSpec((d, rp.shape[1]), lambda i: (0, 0)),
            pl.BlockSpec((cbt.shape[0], k), lambda i: (0, 0)),
            pl.BlockSpec((1, k), lambda i: (0, 0)),
        ],
        out_specs=pl.BlockSpec((1, 1, block_rows), lambda i: (i, 0, 0)),
        out_shape=jax.ShapeDtypeStruct((nb, 1, block_rows), jnp.int32),
        scratch_shapes=[
            pltpu.VMEM((cbt.shape[0], k), jnp.float32),
            pltpu.VMEM((1, k), jnp.float32),
        ],
    )(x2, rp, cbt, iota)
    return out.reshape(bn)


def kernel(x, random_projection, codebook):
    b, n, d = x.shape
    x2 = x.reshape(b * n, d)
    cbt = codebook.T
    idx = _rpq(x2, random_projection, cbt)
    return idx.reshape(b, n)


# speedup vs baseline: 1.1371x; 1.1371x over previous
import functools

import jax
import jax.numpy as jnp
from jax.experimental import pallas as pl
from jax.experimental.pallas import tpu as pltpu


_EPS = 1e-12


def _rpq_body(x_ref, rp_ref, cbt_ref, iota_ref, out_ref, cbn_ref, cbsq_ref):
    i = pl.program_id(0)

    @pl.when(i == 0)
    def _init():
        cbt = cbt_ref[...]
        n = jnp.sqrt(jnp.sum(cbt * cbt, axis=0, keepdims=True))
        cbn = cbt / jnp.maximum(n, _EPS)
        cbn_ref[...] = cbn
        cbsq_ref[...] = jnp.sum(cbn * cbn, axis=0, keepdims=True)

    def matmul_stage(sl):
        proj = jnp.dot(x_ref[sl, :], rp_ref[...],
                       preferred_element_type=jnp.float32)
        norm = jnp.sqrt(jnp.sum(proj * proj, axis=1, keepdims=True))
        projn = proj / jnp.maximum(norm, _EPS)
        x_sq = jnp.sum(projn * projn, axis=1, keepdims=True)
        cm2 = jnp.dot(projn * (-2.0), cbn_ref[...],
                      preferred_element_type=jnp.float32)
        return x_sq, cm2

    def epilogue(x_sq, cm2, sl):
        d2 = (cbsq_ref[...] + x_sq) + cm2
        m2 = jnp.min(d2, axis=1, keepdims=True)
        s = jnp.sqrt(jnp.maximum(m2, 0.0))
        base = s * s
        bi = jax.lax.bitcast_convert_type(base, jnp.int32)
        x_hi = base
        for j in range(1, 7):
            xj = jax.lax.bitcast_convert_type(bi + j, jnp.float32)
            x_hi = jnp.where(jnp.sqrt(xj) == s, xj, x_hi)
        idx = jnp.min(jnp.where(d2 <= x_hi, iota_ref[...], jnp.inf), axis=1)
        out_ref[0, 0, sl] = idx.astype(jnp.int32)

    h = x_ref.shape[0] // 2
    sl1, sl2 = slice(0, h), slice(h, 2 * h)
    r1 = matmul_stage(sl1)
    r2 = matmul_stage(sl2)
    epilogue(*r1, sl1)
    epilogue(*r2, sl2)


@functools.partial(jax.jit, static_argnames=())
def _rpq(x2, rp, cbt):
    bn, d = x2.shape
    k = cbt.shape[1]
    block_rows = 1024
    nb = bn // block_rows
    iota = jnp.arange(k, dtype=jnp.float32).reshape(1, k)
    out = pl.pallas_call(
        _rpq_body,
        grid=(nb,),
        in_specs=[
            pl.BlockSpec((block_rows, d), lambda i: (i, 0)),
            pl.BlockSpec((d, rp.shape[1]), lambda i: (0, 0)),
            pl.BlockSpec((cbt.shape[0], k), lambda i: (0, 0)),
            pl.BlockSpec((1, k), lambda i: (0, 0)),
        ],
        out_specs=pl.BlockSpec((1, 1, block_rows), lambda i: (i, 0, 0)),
        out_shape=jax.ShapeDtypeStruct((nb, 1, block_rows), jnp.int32),
        scratch_shapes=[
            pltpu.VMEM((cbt.shape[0], k), jnp.float32),
            pltpu.VMEM((1, k), jnp.float32),
        ],
    )(x2, rp, cbt, iota)
    return out.reshape(bn)


def kernel(x, random_projection, codebook):
    b, n, d = x.shape
    x2 = x.reshape(b * n, d)
    cbt = codebook.T
    idx = _rpq(x2, random_projection, cbt)
    return idx.reshape(b, n)
